# baseline (speedup 1.0000x reference)
import functools

import jax
import jax.numpy as jnp
from jax.experimental import pallas as pl
from jax.experimental.pallas import tpu as pltpu


def _mlp_softmax_kernel(xa_ref, xb_ref,
                        w1_ref, b1_ref, p_ref, b2_ref, w3_ref, b3_ref,
                        w4_ref, b4_ref, b5_ref,
                        o_ref):
    b1c = b1_ref[...].T
    b2c = b2_ref[...].T
    b3c = b3_ref[...].T
    b4c = b4_ref[...].T
    w2 = p_ref[0:50, 0:10]
    w5t = p_ref[56:61, :]

    def half(x, o_ref, base):
        h = jax.lax.dot_general(
            w1_ref[...], x, (((1,), (1,)), ((), ())),
            preferred_element_type=jnp.float32)
        h = jnp.maximum(h + b1c, 0.0)

        def lin_relu(w, bc, h):
            y = jnp.dot(w, h, preferred_element_type=jnp.float32) + bc
            return jnp.maximum(y, 0.0)

        h = lin_relu(w2, b2c, h)
        h = lin_relu(w3_ref[...], b3c, h)
        h = lin_relu(w4_ref[...], b4c, h)

        logits = jax.lax.dot_general(
            h, w5t, (((0,), (0,)), ((), ())),
            preferred_element_type=jnp.float32) + b5_ref[...]

        m = jnp.max(logits, axis=1, keepdims=True)
        e = jnp.exp(logits - m)
        denom = jnp.sum(e, axis=1, keepdims=True)
        res = e * pl.reciprocal(denom, approx=False)
        o_ref[pl.ds(base, x.shape[0]), :] = res.astype(o_ref.dtype)

    hb = xa_ref.shape[0]
    half(xa_ref[...], o_ref, 0)
    half(xb_ref[...], o_ref, hb)


@functools.partial(jax.jit, static_argnames=("block_b",))
def _forward(x, w1, b1, w2, b2, w3, b3, w4, b4, w5, b5, *, block_b=16384):
    B, D_in = x.shape
    out_dim = w5.shape[0]

    block_b = min(block_b, max(256, ((B + 255) // 256) * 256))
    B_pad = ((B + block_b - 1) // block_b) * block_b
    if B_pad != B:
        x = jnp.pad(x, ((0, B_pad - B), (0, 0)))

    brow = [b.reshape(1, -1) for b in (b1, b2, b3, b4, b5)]

    p = jnp.concatenate(
        [jnp.pad(w2, ((0, 6), (0, out_dim - 10))),
         jnp.pad(w5.T, ((0, 3), (0, 0)))],
        axis=0)

    def full_spec(shape):
        return pl.BlockSpec(shape, lambda i: (0, 0))

    grid = (B_pad // block_b,)

    flops = 2 * B_pad * (D_in * 10 + 10 * 50 + 50 * 10 + 10 * 5 + 5 * out_dim)
    param_bytes = sum(int(v.size) * 4
                      for v in (w1, w2, w3, w4, w5, b1, b2, b3, b4, b5))
    bytes_accessed = B_pad * (D_in + out_dim) * 4 + param_bytes

    half_b = block_b // 2
    operands = [x, x,
                w1, brow[0], p, brow[1], w3, brow[2], w4, brow[3],
                brow[4]]
    in_specs = [pl.BlockSpec((half_b, D_in), lambda i: (2 * i, 0)),
                pl.BlockSpec((half_b, D_in), lambda i: (2 * i + 1, 0))]
    for v in operands[2:]:
        in_specs.append(full_spec(v.shape))

    out = pl.pallas_call(
        _mlp_softmax_kernel,
        out_shape=jax.ShapeDtypeStruct((B_pad, out_dim), jnp.float32),
        grid_spec=pltpu.PrefetchScalarGridSpec(
            num_scalar_prefetch=0,
            grid=grid,
            in_specs=in_specs,
            out_specs=pl.BlockSpec((block_b, out_dim), lambda i: (i, 0)),
        ),
        compiler_params=pltpu.CompilerParams(
            dimension_semantics=("parallel",),
        ),
        cost_estimate=pl.CostEstimate(
            flops=flops,
            transcendentals=B_pad * out_dim,
            bytes_accessed=bytes_accessed),
    )(*operands)

    return out[:B]


def kernel(x, w1, b1, w2, b2, w3, b3, w4, b4, w5, b5):
    return _forward(x, w1, b1, w2, b2, w3, b3, w4, b4, w5, b5)


# speedup vs baseline: 1.0159x; 1.0159x over previous
import functools

import jax
import jax.numpy as jnp
from jax.experimental import pallas as pl
from jax.experimental.pallas import tpu as pltpu


def _mlp_softmax_kernel(xa_ref, xb_ref,
                        w1_ref, b1_ref, p_ref, b2_ref, w3_ref, b3_ref,
                        w4_ref, b4_ref, b5_ref,
                        o_ref):
    b1c = b1_ref[...].T
    b2c = b2_ref[...].T
    b3c = b3_ref[...].T
    b4c = b4_ref[...].T
    w2 = p_ref[0:50, 0:10]
    w5t = p_ref[56:61, :]

    def half(x, o_ref, base):
        h = jax.lax.dot_general(
            w1_ref[...], x, (((1,), (1,)), ((), ())),
            preferred_element_type=jnp.float32)
        h = jnp.maximum(h + b1c, 0.0)

        def lin_relu(w, bc, h):
            y = jnp.dot(w, h, preferred_element_type=jnp.float32) + bc
            return jnp.maximum(y, 0.0)

        h = lin_relu(w2, b2c, h)
        h = lin_relu(w3_ref[...], b3c, h)
        h = lin_relu(w4_ref[...], b4c, h)

        logits = jax.lax.dot_general(
            h, w5t, (((0,), (0,)), ((), ())),
            preferred_element_type=jnp.float32) + b5_ref[...]

        m = jnp.max(logits, axis=1, keepdims=True)
        e = jnp.exp(logits - m)
        denom = jnp.sum(e, axis=1, keepdims=True)
        res = e * pl.reciprocal(denom, approx=False)
        o_ref[pl.ds(base, x.shape[0]), :] = res.astype(o_ref.dtype)

    hb = xa_ref.shape[0]
    half(xa_ref[...], o_ref, 0)
    half(xb_ref[...], o_ref, hb)


@functools.partial(jax.jit, static_argnames=("block_b",))
def _forward(x, w1, b1, w2, b2, w3, b3, w4, b4, w5, b5, *, block_b=8192):
    B, D_in = x.shape
    out_dim = w5.shape[0]

    block_b = min(block_b, max(256, ((B + 255) // 256) * 256))
    B_pad = ((B + block_b - 1) // block_b) * block_b
    if B_pad != B:
        x = jnp.pad(x, ((0, B_pad - B), (0, 0)))

    brow = [b.reshape(1, -1) for b in (b1, b2, b3, b4, b5)]

    p = jnp.concatenate(
        [jnp.pad(w2, ((0, 6), (0, out_dim - 10))),
         jnp.pad(w5.T, ((0, 3), (0, 0)))],
        axis=0)

    def full_spec(shape):
        return pl.BlockSpec(shape, lambda i: (0, 0))

    grid = (B_pad // block_b,)

    flops = 2 * B_pad * (D_in * 10 + 10 * 50 + 50 * 10 + 10 * 5 + 5 * out_dim)
    param_bytes = sum(int(v.size) * 4
                      for v in (w1, w2, w3, w4, w5, b1, b2, b3, b4, b5))
    bytes_accessed = B_pad * (D_in + out_dim) * 4 + param_bytes

    half_b = block_b // 2
    operands = [x, x,
                w1, brow[0], p, brow[1], w3, brow[2], w4, brow[3],
                brow[4]]
    in_specs = [pl.BlockSpec((half_b, D_in), lambda i: (2 * i, 0)),
                pl.BlockSpec((half_b, D_in), lambda i: (2 * i + 1, 0))]
    for v in operands[2:]:
        in_specs.append(full_spec(v.shape))

    out = pl.pallas_call(
        _mlp_softmax_kernel,
        out_shape=jax.ShapeDtypeStruct((B_pad, out_dim), jnp.float32),
        grid_spec=pltpu.PrefetchScalarGridSpec(
            num_scalar_prefetch=0,
            grid=grid,
            in_specs=in_specs,
            out_specs=pl.BlockSpec((block_b, out_dim), lambda i: (i, 0)),
        ),
        compiler_params=pltpu.CompilerParams(
            dimension_semantics=("parallel",),
        ),
        cost_estimate=pl.CostEstimate(
            flops=flops,
            transcendentals=B_pad * out_dim,
            bytes_accessed=bytes_accessed),
    )(*operands)

    return out[:B]


def kernel(x, w1, b1, w2, b2, w3, b3, w4, b4, w5, b5):
    return _forward(x, w1, b1, w2, b2, w3, b3, w4, b4, w5, b5)
